# baseline (speedup 1.0000x reference)
import functools
import math

import jax
import jax.numpy as jnp
from jax import lax
from jax.experimental import pallas as pl
from jax.experimental.pallas import tpu as pltpu
from jax.experimental.pallas import tpu_sc as plsc

D_MODEL = 128
SCALE = math.sqrt(128.0)
NUM_CORES = 2
NUM_SUBCORES = 16
NUM_WORKERS = NUM_CORES * NUM_SUBCORES
SEQ = 50
BPS = 4
ROWS = BPS * SEQ
HALF = ROWS // 2
IPAD = 104


@functools.partial(jax.jit, static_argnames=("batches",))
def _embed_sc(idx, table, batches):
    b_per_w = batches // NUM_WORKERS
    nsteps = b_per_w // BPS

    @functools.partial(
        pl.kernel,
        out_type=jax.ShapeDtypeStruct((batches, SEQ, D_MODEL), jnp.float32),
        mesh=plsc.VectorSubcoreMesh(core_axis_name="c", subcore_axis_name="s"),
        scratch_types=[
            pltpu.VMEM((2 * nsteps, IPAD), jnp.int32),
            pltpu.VMEM((4, ROWS, D_MODEL), jnp.float32),
            [pltpu.SemaphoreType.DMA] * 4,
            [pltpu.SemaphoreType.DMA] * 4,
        ],
    )
    def k(idx_hbm, table_hbm, out_hbm, idx_v, bufs, gsems, osems):
        wid = lax.axis_index("s") * NUM_CORES + lax.axis_index("c")
        pltpu.sync_copy(idx_hbm.at[wid], idx_v)
        base = wid * b_per_w

        def scale(buf):
            def body(q, c2):
                r0 = q * 10
                for r in range(10):
                    for j in range(D_MODEL // 16):
                        sl = pl.ds(j * 16, 16)
                        buf[r0 + r, sl] = buf[r0 + r, sl] * SCALE
                return c2

            lax.fori_loop(0, ROWS // 10, body, 0)

        def gather_start(g, buf, sem):
            for h in range(2):
                pltpu.async_copy(
                    table_hbm.at[idx_v.at[2 * g + h, pl.ds(0, HALF)]],
                    buf.at[pl.ds(h * HALF, HALF)], sem)

        def gather_wait(g, buf, sem):
            for h in range(2):
                pltpu.make_async_copy(
                    table_hbm.at[idx_v.at[2 * g + h, pl.ds(0, HALF)]],
                    buf.at[pl.ds(h * HALF, HALF)], sem).wait()

        def put_start(g, buf, sem):
            b0 = base + g * BPS
            for b in range(BPS):
                pltpu.async_copy(
                    buf.at[pl.ds(b * SEQ, SEQ)], out_hbm.at[b0 + b], sem)

        def put_wait(g, buf, sem):
            b0 = base + g * BPS
            for b in range(BPS):
                pltpu.make_async_copy(
                    buf.at[pl.ds(b * SEQ, SEQ)], out_hbm.at[b0 + b],
                    sem).wait()

        gather_start(0, bufs.at[0], gsems[0])
        gather_start(1, bufs.at[1], gsems[1])

        def quad(q, carry):
            g0 = q * 4
            for i in range(4):
                g = g0 + i
                buf = bufs.at[i]
                gather_wait(g, buf, gsems[i])

                @pl.when(g >= 2)
                def _():
                    put_wait(g - 2, bufs.at[(i + 2) % 4], osems[(i + 2) % 4])

                @pl.when(g + 2 < nsteps)
                def _():
                    gather_start(
                        g + 2, bufs.at[(i + 2) % 4], gsems[(i + 2) % 4])

                scale(buf)
                put_start(g, buf, osems[i])
            return carry

        lax.fori_loop(0, nsteps // 4, quad, 0)
        put_wait(nsteps - 2, bufs.at[2], osems[2])
        put_wait(nsteps - 1, bufs.at[3], osems[3])

    return k(idx, table)


def kernel(x, word_emb):
    batches = x.shape[0]
    b_per_w = batches // NUM_WORKERS
    nsteps = b_per_w // BPS
    xr = x.reshape(NUM_WORKERS, 2 * nsteps, HALF).astype(jnp.int32)
    idx = jnp.pad(xr, ((0, 0), (0, 0), (0, IPAD - HALF)))
    return _embed_sc(idx, word_emb, batches)


# speedup vs baseline: 1.0050x; 1.0050x over previous
import functools
import math

import jax
import jax.numpy as jnp
from jax import lax
from jax.experimental import pallas as pl
from jax.experimental.pallas import tpu as pltpu
from jax.experimental.pallas import tpu_sc as plsc

D_MODEL = 128
SCALE = math.sqrt(128.0)
NUM_CORES = 2
NUM_SUBCORES = 16
NUM_WORKERS = NUM_CORES * NUM_SUBCORES
SEQ = 50
BPS = 4
ROWS = BPS * SEQ
HALF = ROWS // 2
IPAD = 104


@functools.partial(jax.jit, static_argnames=("batches",))
def _embed_sc(idx, table, batches):
    b_per_w = batches // NUM_WORKERS
    nsteps = b_per_w // BPS

    @functools.partial(
        pl.kernel,
        out_type=jax.ShapeDtypeStruct((batches, SEQ, D_MODEL), jnp.float32),
        mesh=plsc.VectorSubcoreMesh(core_axis_name="c", subcore_axis_name="s"),
        scratch_types=[
            pltpu.VMEM((2 * nsteps, IPAD), jnp.int32),
            pltpu.VMEM((4, ROWS, D_MODEL), jnp.float32),
            [pltpu.SemaphoreType.DMA] * 4,
            [pltpu.SemaphoreType.DMA] * 4,
        ],
    )
    def k(idx_hbm, table_hbm, out_hbm, idx_v, bufs, gsems, osems):
        wid = lax.axis_index("s") * NUM_CORES + lax.axis_index("c")
        pltpu.sync_copy(idx_hbm.at[wid], idx_v)
        base = wid * b_per_w

        def scale(buf):
            def body(q, c2):
                r0 = q * 5
                for r in range(5):
                    for j in range(D_MODEL // 16):
                        sl = pl.ds(j * 16, 16)
                        buf[r0 + r, sl] = buf[r0 + r, sl] * SCALE
                return c2

            lax.fori_loop(0, ROWS // 5, body, 0)

        def gather_start(g, buf, sem):
            for h in range(2):
                pltpu.async_copy(
                    table_hbm.at[idx_v.at[2 * g + h, pl.ds(0, HALF)]],
                    buf.at[pl.ds(h * HALF, HALF)], sem)

        def gather_wait(g, buf, sem):
            for h in range(2):
                pltpu.make_async_copy(
                    table_hbm.at[idx_v.at[2 * g + h, pl.ds(0, HALF)]],
                    buf.at[pl.ds(h * HALF, HALF)], sem).wait()

        def put_start(g, buf, sem):
            b0 = base + g * BPS
            for b in range(BPS):
                pltpu.async_copy(
                    buf.at[pl.ds(b * SEQ, SEQ)], out_hbm.at[b0 + b], sem)

        def put_wait(g, buf, sem):
            b0 = base + g * BPS
            for b in range(BPS):
                pltpu.make_async_copy(
                    buf.at[pl.ds(b * SEQ, SEQ)], out_hbm.at[b0 + b],
                    sem).wait()

        gather_start(0, bufs.at[0], gsems[0])
        gather_start(1, bufs.at[1], gsems[1])

        def quad(q, carry):
            g0 = q * 4
            for i in range(4):
                g = g0 + i
                buf = bufs.at[i]
                gather_wait(g, buf, gsems[i])

                @pl.when(g >= 2)
                def _():
                    put_wait(g - 2, bufs.at[(i + 2) % 4], osems[(i + 2) % 4])

                @pl.when(g + 2 < nsteps)
                def _():
                    gather_start(
                        g + 2, bufs.at[(i + 2) % 4], gsems[(i + 2) % 4])

                scale(buf)
                put_start(g, buf, osems[i])
            return carry

        lax.fori_loop(0, nsteps // 4, quad, 0)
        put_wait(nsteps - 2, bufs.at[2], osems[2])
        put_wait(nsteps - 1, bufs.at[3], osems[3])

    return k(idx, table)


def kernel(x, word_emb):
    batches = x.shape[0]
    b_per_w = batches // NUM_WORKERS
    nsteps = b_per_w // BPS
    xr = x.reshape(NUM_WORKERS, 2 * nsteps, HALF).astype(jnp.int32)
    idx = jnp.pad(xr, ((0, 0), (0, 0), (0, IPAD - HALF)))
    return _embed_sc(idx, word_emb, batches)
